# baseline (speedup 1.0000x reference)
import functools

import jax
import jax.numpy as jnp
from jax import lax
from jax.experimental import pallas as pl
from jax.experimental.pallas import tpu as pltpu
from jax.experimental.pallas import tpu_sc as plsc

N = 10000
E = 320000
C = 4
NPAD = 10240
NT = 32
RANGE = NPAD // NT
BLK = RANGE // 4
NBLK = NPAD // BLK
MAINCAP = 12288
SUBCAP = 4096
LSLACK = 64
CH = 2000
GB = 32
GB1 = 64
HD = 128
HD1 = 64
NEG = -3.0e38
SENT = 1 << 30


def _popcount(m):
    return plsc.all_reduce_population_count(m)[0]


def _process_block(x_hbm, agg_hbm, ssrc, slidx, accs, rows, gsems, cs, gb,
                   packed, gbatch, chunks):
    if packed:
        negv = plsc.bitcast(jnp.full((32,), NEG, jnp.bfloat16), jnp.int32)
    else:
        negv = jnp.full((16,), NEG, jnp.float32)

    def _vmax(a, b):
        if packed:
            return plsc.bitcast(jnp.maximum(
                plsc.bitcast(a, jnp.bfloat16),
                plsc.bitcast(b, jnp.bfloat16)), jnp.int32)
        return jnp.maximum(a, b)

    def initacc(i, _):
        for j in range(chunks):
            for a in accs:
                a[i, pl.ds(j * 16, 16)] = negv
        return 0
    lax.fori_loop(0, RANGE, initacc, 0)

    nb = (cs + gbatch - 1) // gbatch

    def _fire(k, b):
        pltpu.async_copy(x_hbm.at[ssrc.at[pl.ds(k * gbatch, gbatch)]],
                         rows[b], gsems[b])

    def _wait(b):
        pltpu.make_async_copy(
            x_hbm.at[ssrc.at[pl.ds(0, gbatch)]], rows[b], gsems[b]).wait()

    @pl.when(nb > 0)
    def _():
        _fire(0, 0)

    def gpair(p, _):
        for b in (0, 1):
            k = 2 * p + b

            @pl.when(k < nb)
            def _():
                _wait(b)

                @pl.when(k + 1 < nb)
                def _():
                    _fire(k + 1, b ^ 1)

                rb = rows[b]
                for g in range(gbatch // 16):
                    lv = slidx[pl.ds(k * gbatch + g * 16, 16)]
                    for e in range(16):
                        li = lv[e]
                        acc = accs[e % len(accs)]
                        for j in range(chunks):
                            sl = pl.ds(j * 16, 16)
                            acc[li, sl] = _vmax(acc[li, sl],
                                                rb[g * 16 + e, sl])
        return 0
    lax.fori_loop(0, (nb + 1) // 2, gpair, 0)

    if packed:
        zero = jnp.zeros((32,), jnp.bfloat16)
        thresh = jnp.full((32,), -1e37, jnp.bfloat16)
    else:
        zero = jnp.zeros((16,), jnp.float32)
        thresh = jnp.full((16,), -1e37, jnp.float32)

    def fix(i, _):
        for j in range(chunks):
            sl = pl.ds(j * 16, 16)
            v = accs[0][i, sl]
            if len(accs) > 1:
                v = _vmax(v, accs[1][i, sl])
            if packed:
                vb = plsc.bitcast(v, jnp.bfloat16)
                accs[0][i, sl] = plsc.bitcast(
                    jnp.where(vb < thresh, zero, vb), jnp.int32)
            else:
                accs[0][i, sl] = jnp.where(v < thresh, zero, v)
        return 0
    lax.fori_loop(0, RANGE, fix, 0)

    for t in range(C):
        pltpu.sync_copy(
            accs[0].at[pl.ds(t * BLK, BLK)],
            agg_hbm.at[pl.ds(t * NPAD + gb * BLK, BLK)])


@functools.cache
def _build_conv1():
    mesh = plsc.VectorSubcoreMesh(
        core_axis_name="c", subcore_axis_name="s", num_cores=2,
        num_subcores=16)
    out_types = (
        jax.ShapeDtypeStruct((C * NPAD, HD), jnp.float32),
        jax.ShapeDtypeStruct((NBLK, SUBCAP), jnp.int32),
        jax.ShapeDtypeStruct((NBLK, SUBCAP), jnp.int32),
        jax.ShapeDtypeStruct((NT, 16), jnp.int32),
    )
    scratch = [
        pltpu.VMEM((2 * CH,), jnp.int32),
        pltpu.VMEM((MAINCAP + 16,), jnp.int32),
        pltpu.VMEM((SUBCAP + LSLACK,), jnp.int32),
        pltpu.VMEM((SUBCAP + LSLACK,), jnp.int32),
        pltpu.VMEM((RANGE + 1, HD), jnp.float32),
        pltpu.VMEM((GB, HD), jnp.float32),
        pltpu.VMEM((GB, HD), jnp.float32),
        pltpu.VMEM((16,), jnp.int32),
        pltpu.SemaphoreType.DMA,
        pltpu.SemaphoreType.DMA,
        pltpu.SemaphoreType.DMA,
        pltpu.SemaphoreType.DMA,
    ]

    @functools.partial(
        pl.kernel, out_type=out_types, mesh=mesh, scratch_types=scratch,
        compiler_params=pltpu.CompilerParams(needs_layout_passes=False))
    def conv1(x_hbm, ew_hbm,
              agg_hbm, srcl_hbm, lidxl_hbm, cnts_hbm,
              ew_st, mlist, ssrc, slidx,
              accA, rows0, rows1, cntv,
              st_sem0, st_sem1, g_sem0, g_sem1):
        tid = lax.axis_index("s") * 2 + lax.axis_index("c")
        tbase4 = tid * (RANGE * 4)
        iota = lax.iota(jnp.int32, 16)

        def pre_s(i, _):
            ssrc[pl.ds(i * 16, 16)] = (iota + i * 16) & 8191
            return 0
        lax.fori_loop(0, (SUBCAP + LSLACK) // 16, pre_s, 0)
        sent = jnp.full((16,), SENT, jnp.int32)

        def pre_m(i, _):
            mlist[pl.ds(i * 16, 16)] = sent
            return 0
        lax.fori_loop(0, (MAINCAP + 16) // 16, pre_m, 0)

        stage_sems = (st_sem0, st_sem1)

        def _fire_stage(ci, b):
            pltpu.async_copy(ew_hbm.at[pl.ds(ci * CH, CH)],
                             ew_st.at[pl.ds(b * CH, CH)], stage_sems[b])

        def _wait_stage(b):
            pltpu.make_async_copy(ew_hbm.at[pl.ds(0, CH)],
                                  ew_st.at[pl.ds(b * CH, CH)],
                                  stage_sems[b]).wait()

        NCH = E // CH
        _fire_stage(0, 0)

        def scan_pair(p, cnt):
            for b in (0, 1):
                ci = 2 * p + b
                _wait_stage(b)

                @pl.when(ci + 1 < NCH)
                def _():
                    _fire_stage(ci + 1, b ^ 1)

                def sub(i, cnt):
                    w = ew_st[pl.ds(b * CH + i * 16, 16)]
                    rel = (w >> 14) - tbase4
                    m = (rel >= 0) & (rel < RANGE * 4)
                    plsc.store_compressed(mlist.at[pl.ds(cnt, 16)], w, mask=m)
                    return jnp.minimum(cnt + _popcount(m), MAINCAP)
                cnt = lax.fori_loop(0, CH // 16, sub, cnt)
            return cnt
        cnt_main = lax.fori_loop(0, NCH // 2, scan_pair, jnp.int32(0))

        rows = (rows0, rows1)
        gsems = (g_sem0, g_sem1)
        dummy = jnp.full((16,), RANGE, jnp.int32)

        def do_block(blk, cntvec):
            gb = tid * 4 + blk
            lo = tbase4 + blk * 4 * BLK

            def subc(i, cs):
                w = mlist[pl.ds(i * 16, 16)]
                cc = (w >> 14) - lo
                m = (cc >= 0) & (cc < 4 * BLK)
                lidx = (cc & 3) * BLK + (cc >> 2)
                s = w & 16383
                plsc.store_compressed(ssrc.at[pl.ds(cs, 16)], s, mask=m)
                plsc.store_compressed(slidx.at[pl.ds(cs, 16)], lidx, mask=m)
                return jnp.minimum(cs + _popcount(m), SUBCAP)
            cs = lax.fori_loop(0, (cnt_main + 15) // 16, subc, jnp.int32(0))

            slidx[pl.ds(cs, 16)] = dummy
            slidx[pl.ds(cs + 16, 16)] = dummy
            pltpu.sync_copy(ssrc.at[pl.ds(0, SUBCAP)], srcl_hbm.at[gb])
            pltpu.sync_copy(slidx.at[pl.ds(0, SUBCAP)], lidxl_hbm.at[gb])

            _process_block(x_hbm, agg_hbm, ssrc, slidx, (accA,),
                           rows, gsems, cs, gb, packed=False, gbatch=GB,
                           chunks=HD // 16)
            return jnp.where(iota == blk, cs, cntvec)

        cntvec = lax.fori_loop(0, 4, do_block, jnp.zeros((16,), jnp.int32))
        cntv[...] = cntvec
        pltpu.sync_copy(cntv, cnts_hbm.at[tid])

    return conv1


@functools.cache
def _build_conv2():
    mesh = plsc.VectorSubcoreMesh(
        core_axis_name="c", subcore_axis_name="s", num_cores=2,
        num_subcores=16)
    out_types = jax.ShapeDtypeStruct((C * NPAD, HD), jnp.int32)
    scratch = [
        pltpu.VMEM((SUBCAP + LSLACK,), jnp.int32),
        pltpu.VMEM((SUBCAP + LSLACK,), jnp.int32),
        pltpu.VMEM((RANGE + 1, HD), jnp.int32),
        pltpu.VMEM((RANGE + 1, HD), jnp.int32),
        pltpu.VMEM((GB, HD), jnp.int32),
        pltpu.VMEM((GB, HD), jnp.int32),
        pltpu.VMEM((16,), jnp.int32),
        pltpu.SemaphoreType.DMA,
        pltpu.SemaphoreType.DMA,
    ]

    @functools.partial(
        pl.kernel, out_type=out_types, mesh=mesh, scratch_types=scratch,
        compiler_params=pltpu.CompilerParams(needs_layout_passes=False))
    def conv2(h_hbm, srcl_hbm, lidxl_hbm, cnts_hbm, agg_hbm,
              ssrc, slidx, accA, accB, rows0, rows1, cntv,
              g_sem0, g_sem1):
        tid = lax.axis_index("s") * 2 + lax.axis_index("c")
        iota = lax.iota(jnp.int32, 16)
        pltpu.sync_copy(cnts_hbm.at[tid], cntv)
        cv = cntv[pl.ds(0, 16)]
        rows = (rows0, rows1)
        gsems = (g_sem0, g_sem1)

        def do_block(blk, _):
            gb = tid * 4 + blk
            pltpu.sync_copy(srcl_hbm.at[gb], ssrc.at[pl.ds(0, SUBCAP)])
            pltpu.sync_copy(lidxl_hbm.at[gb], slidx.at[pl.ds(0, SUBCAP)])
            cs = jnp.sum(jnp.where(iota == blk, cv, 0))
            _process_block(h_hbm, agg_hbm, ssrc, slidx, (accA, accB),
                           rows, gsems, cs, gb, packed=True, gbatch=GB,
                           chunks=HD // 16)
            return 0
        lax.fori_loop(0, 4, do_block, 0)

    return conv2


def _dense1(x, agg, wt, cwt, b2d):
    R = 512

    def body(x_ref, agg_ref, w_ref, cw_ref, b_ref, o_ref, op_ref):
        a = jnp.dot(x_ref[...], w_ref[...],
                    preferred_element_type=jnp.float32)
        for c in range(C):
            a = a + jnp.dot(agg_ref[c], cw_ref[c],
                            preferred_element_type=jnp.float32)
        h = jnp.maximum(a + b_ref[0:1, :], 0.0)
        o_ref[...] = h
        u0 = lax.bitcast_convert_type(h[:, :HD], jnp.int32)
        u1 = lax.bitcast_convert_type(h[:, HD:], jnp.int32)
        r0 = (u0 + 0x7FFF + ((u0 >> 16) & 1)) >> 16
        r1 = (u1 + 0x7FFF + ((u1 >> 16) & 1)) >> 16
        op_ref[...] = (r0 & 0xFFFF) | (r1 << 16)

    return pl.pallas_call(
        body,
        grid=(NPAD // R,),
        in_specs=[
            pl.BlockSpec((R, 128), lambda i: (i, 0)),
            pl.BlockSpec((C, R, 128), lambda i: (0, i, 0)),
            pl.BlockSpec((128, 256), lambda i: (0, 0)),
            pl.BlockSpec((C, 128, 256), lambda i: (0, 0, 0)),
            pl.BlockSpec((8, 256), lambda i: (0, 0)),
        ],
        out_specs=[pl.BlockSpec((R, 256), lambda i: (i, 0)),
                   pl.BlockSpec((R, HD), lambda i: (i, 0))],
        out_shape=[jax.ShapeDtypeStruct((NPAD, 256), jnp.float32),
                   jax.ShapeDtypeStruct((NPAD, HD), jnp.int32)],
    )(x, agg, wt, cwt, b2d)


def _dense2(x, agg, wt, cwt, b2d):
    R = 512

    def body(x_ref, agg_ref, w_ref, cw_ref, b_ref, o_ref):
        a = jnp.dot(x_ref[...], w_ref[...],
                    preferred_element_type=jnp.float32)
        for c in range(C):
            w = agg_ref[c]
            low = lax.bitcast_convert_type(w << 16, jnp.float32)
            high = lax.bitcast_convert_type(w & ~0xFFFF, jnp.float32)
            a = a + jnp.dot(low, cw_ref[c][:HD],
                            preferred_element_type=jnp.float32)
            a = a + jnp.dot(high, cw_ref[c][HD:],
                            preferred_element_type=jnp.float32)
        o_ref[...] = jax.nn.sigmoid(a + b_ref[0:1, :] - 10.0)

    return pl.pallas_call(
        body,
        grid=(NPAD // R,),
        in_specs=[
            pl.BlockSpec((R, 256), lambda i: (i, 0)),
            pl.BlockSpec((C, R, HD), lambda i: (0, i, 0)),
            pl.BlockSpec((256, 128), lambda i: (0, 0)),
            pl.BlockSpec((C, 256, 128), lambda i: (0, 0, 0)),
            pl.BlockSpec((8, 128), lambda i: (0, 0)),
        ],
        out_specs=pl.BlockSpec((R, 128), lambda i: (i, 0)),
        out_shape=jax.ShapeDtypeStruct((NPAD, 128), jnp.float32),
    )(x, agg, wt, cwt, b2d)


def kernel(x, edge_index, edge_type, conv1_w, conv2_w,
           lin1_w, lin1_b, lin2_w, lin2_b):
    src = edge_index[0].astype(jnp.int32)
    dst = edge_index[1].astype(jnp.int32)
    typ = edge_type.astype(jnp.int32)
    ew = src | ((dst * 4 + typ) << 14)

    agg1f, srcl, lidxl, cnts = _build_conv1()(x, ew)
    agg1 = agg1f.reshape(C, NPAD, 128)

    x_pad = jnp.pad(x, ((0, NPAD - N), (0, 0)))
    b1 = jnp.broadcast_to(lin1_b[None, :], (8, 256))
    h1, h1p = _dense1(x_pad, agg1, lin1_w.T,
                      jnp.transpose(conv1_w, (0, 2, 1)), b1)

    agg2f = _build_conv2()(h1p, srcl, lidxl, cnts)
    agg2 = agg2f.reshape(C, NPAD, HD)

    b2 = jnp.broadcast_to(lin2_b[None, :], (8, 128))
    out = _dense2(h1, agg2, lin2_w.T, jnp.transpose(conv2_w, (0, 2, 1)), b2)
    return out[:N]


# speedup vs baseline: 1.1927x; 1.1927x over previous
import functools

import jax
import jax.numpy as jnp
from jax import lax
from jax.experimental import pallas as pl
from jax.experimental.pallas import tpu as pltpu
from jax.experimental.pallas import tpu_sc as plsc

N = 10000
E = 320000
C = 4
NPAD = 10240
NT = 32
RANGE = NPAD // NT
BLK = RANGE // 4
NBLK = NPAD // BLK
MAINCAP = 12288
SUBCAP = 4096
LSLACK = 64
CH = 2000
GB = 32
GB1 = 64
HD = 128
HD1 = 64
NEG = -3.0e38
SENT = 1 << 30


def _popcount(m):
    return plsc.all_reduce_population_count(m)[0]


def _process_block(x_hbm, agg_hbm, ssrc, slidx, accs, rows, gsems, cs, gb,
                   packed, gbatch, chunks):
    if packed:
        negv = plsc.bitcast(jnp.full((32,), NEG, jnp.bfloat16), jnp.int32)
    else:
        negv = jnp.full((16,), NEG, jnp.float32)

    def _vmax(a, b):
        if packed:
            return plsc.bitcast(jnp.maximum(
                plsc.bitcast(a, jnp.bfloat16),
                plsc.bitcast(b, jnp.bfloat16)), jnp.int32)
        return jnp.maximum(a, b)

    def initacc(i, _):
        for j in range(chunks):
            for a in accs:
                a[i, pl.ds(j * 16, 16)] = negv
        return 0
    lax.fori_loop(0, RANGE, initacc, 0)

    nb = (cs + gbatch - 1) // gbatch

    def _fire(k, b):
        pltpu.async_copy(x_hbm.at[ssrc.at[pl.ds(k * gbatch, gbatch)]],
                         rows[b], gsems[b])

    def _wait(b):
        pltpu.make_async_copy(
            x_hbm.at[ssrc.at[pl.ds(0, gbatch)]], rows[b], gsems[b]).wait()

    @pl.when(nb > 0)
    def _():
        _fire(0, 0)

    def gpair(p, _):
        for b in (0, 1):
            k = 2 * p + b

            @pl.when(k < nb)
            def _():
                _wait(b)

                @pl.when(k + 1 < nb)
                def _():
                    _fire(k + 1, b ^ 1)

                rb = rows[b]
                for g in range(gbatch // 16):
                    lv = slidx[pl.ds(k * gbatch + g * 16, 16)]
                    for e in range(16):
                        li = lv[e]
                        acc = accs[e % len(accs)]
                        for j in range(chunks):
                            sl = pl.ds(j * 16, 16)
                            acc[li, sl] = _vmax(acc[li, sl],
                                                rb[g * 16 + e, sl])
        return 0
    lax.fori_loop(0, (nb + 1) // 2, gpair, 0)

    if packed:
        zero = jnp.zeros((32,), jnp.bfloat16)
        thresh = jnp.full((32,), -1e37, jnp.bfloat16)
    else:
        zero = jnp.zeros((16,), jnp.float32)
        thresh = jnp.full((16,), -1e37, jnp.float32)

    def fix(i, _):
        for j in range(chunks):
            sl = pl.ds(j * 16, 16)
            v = accs[0][i, sl]
            if len(accs) > 1:
                v = _vmax(v, accs[1][i, sl])
            if packed:
                vb = plsc.bitcast(v, jnp.bfloat16)
                accs[0][i, sl] = plsc.bitcast(
                    jnp.where(vb < thresh, zero, vb), jnp.int32)
            else:
                accs[0][i, sl] = jnp.where(v < thresh, zero, v)
        return 0
    lax.fori_loop(0, RANGE, fix, 0)

    for t in range(C):
        pltpu.sync_copy(
            accs[0].at[pl.ds(t * BLK, BLK)],
            agg_hbm.at[pl.ds(t * NPAD + gb * BLK, BLK)])


@functools.cache
def _build_conv1():
    mesh = plsc.VectorSubcoreMesh(
        core_axis_name="c", subcore_axis_name="s", num_cores=2,
        num_subcores=16)
    out_types = (
        jax.ShapeDtypeStruct((C * NPAD, HD), jnp.int32),
        jax.ShapeDtypeStruct((NBLK, SUBCAP), jnp.int32),
        jax.ShapeDtypeStruct((NBLK, SUBCAP), jnp.int32),
        jax.ShapeDtypeStruct((NT, 16), jnp.int32),
    )
    scratch = [
        pltpu.VMEM((2 * CH,), jnp.int32),
        pltpu.VMEM((MAINCAP + 16,), jnp.int32),
        pltpu.VMEM((SUBCAP + LSLACK,), jnp.int32),
        pltpu.VMEM((SUBCAP + LSLACK,), jnp.int32),
        pltpu.VMEM((RANGE + 1, HD), jnp.int32),
        pltpu.VMEM((GB, HD), jnp.int32),
        pltpu.VMEM((GB, HD), jnp.int32),
        pltpu.VMEM((16,), jnp.int32),
        pltpu.SemaphoreType.DMA,
        pltpu.SemaphoreType.DMA,
        pltpu.SemaphoreType.DMA,
        pltpu.SemaphoreType.DMA,
    ]

    @functools.partial(
        pl.kernel, out_type=out_types, mesh=mesh, scratch_types=scratch,
        compiler_params=pltpu.CompilerParams(needs_layout_passes=False))
    def conv1(x_hbm, ew_hbm,
              agg_hbm, srcl_hbm, lidxl_hbm, cnts_hbm,
              ew_st, mlist, ssrc, slidx,
              accA, rows0, rows1, cntv,
              st_sem0, st_sem1, g_sem0, g_sem1):
        tid = lax.axis_index("s") * 2 + lax.axis_index("c")
        tbase4 = tid * (RANGE * 4)
        iota = lax.iota(jnp.int32, 16)

        def pre_s(i, _):
            ssrc[pl.ds(i * 16, 16)] = (iota + i * 16) & 8191
            return 0
        lax.fori_loop(0, (SUBCAP + LSLACK) // 16, pre_s, 0)
        sent = jnp.full((16,), SENT, jnp.int32)

        def pre_m(i, _):
            mlist[pl.ds(i * 16, 16)] = sent
            return 0
        lax.fori_loop(0, (MAINCAP + 16) // 16, pre_m, 0)

        stage_sems = (st_sem0, st_sem1)

        def _fire_stage(ci, b):
            pltpu.async_copy(ew_hbm.at[pl.ds(ci * CH, CH)],
                             ew_st.at[pl.ds(b * CH, CH)], stage_sems[b])

        def _wait_stage(b):
            pltpu.make_async_copy(ew_hbm.at[pl.ds(0, CH)],
                                  ew_st.at[pl.ds(b * CH, CH)],
                                  stage_sems[b]).wait()

        NCH = E // CH
        _fire_stage(0, 0)

        def scan_pair(p, cnt):
            for b in (0, 1):
                ci = 2 * p + b
                _wait_stage(b)

                @pl.when(ci + 1 < NCH)
                def _():
                    _fire_stage(ci + 1, b ^ 1)

                def sub(i, cnt):
                    w = ew_st[pl.ds(b * CH + i * 16, 16)]
                    rel = (w >> 14) - tbase4
                    m = (rel >= 0) & (rel < RANGE * 4)
                    plsc.store_compressed(mlist.at[pl.ds(cnt, 16)], w, mask=m)
                    return jnp.minimum(cnt + _popcount(m), MAINCAP)
                cnt = lax.fori_loop(0, CH // 16, sub, cnt)
            return cnt
        cnt_main = lax.fori_loop(0, NCH // 2, scan_pair, jnp.int32(0))

        rows = (rows0, rows1)
        gsems = (g_sem0, g_sem1)
        dummy = jnp.full((16,), RANGE, jnp.int32)

        def do_block(blk, cntvec):
            gb = tid * 4 + blk
            lo = tbase4 + blk * 4 * BLK

            def subc(i, cs):
                w = mlist[pl.ds(i * 16, 16)]
                cc = (w >> 14) - lo
                m = (cc >= 0) & (cc < 4 * BLK)
                lidx = (cc & 3) * BLK + (cc >> 2)
                s = w & 16383
                plsc.store_compressed(ssrc.at[pl.ds(cs, 16)], s, mask=m)
                plsc.store_compressed(slidx.at[pl.ds(cs, 16)], lidx, mask=m)
                return jnp.minimum(cs + _popcount(m), SUBCAP)
            cs = lax.fori_loop(0, (cnt_main + 15) // 16, subc, jnp.int32(0))

            slidx[pl.ds(cs, 16)] = dummy
            slidx[pl.ds(cs + 16, 16)] = dummy
            pltpu.sync_copy(ssrc.at[pl.ds(0, SUBCAP)], srcl_hbm.at[gb])
            pltpu.sync_copy(slidx.at[pl.ds(0, SUBCAP)], lidxl_hbm.at[gb])

            _process_block(x_hbm, agg_hbm, ssrc, slidx, (accA,),
                           rows, gsems, cs, gb, packed=True, gbatch=GB,
                           chunks=HD1 // 16)
            return jnp.where(iota == blk, cs, cntvec)

        cntvec = lax.fori_loop(0, 4, do_block, jnp.zeros((16,), jnp.int32))
        cntv[...] = cntvec
        pltpu.sync_copy(cntv, cnts_hbm.at[tid])

    return conv1


@functools.cache
def _build_conv2():
    mesh = plsc.VectorSubcoreMesh(
        core_axis_name="c", subcore_axis_name="s", num_cores=2,
        num_subcores=16)
    out_types = jax.ShapeDtypeStruct((C * NPAD, HD), jnp.int32)
    scratch = [
        pltpu.VMEM((SUBCAP + LSLACK,), jnp.int32),
        pltpu.VMEM((SUBCAP + LSLACK,), jnp.int32),
        pltpu.VMEM((RANGE + 1, HD), jnp.int32),
        pltpu.VMEM((GB, HD), jnp.int32),
        pltpu.VMEM((GB, HD), jnp.int32),
        pltpu.VMEM((16,), jnp.int32),
        pltpu.SemaphoreType.DMA,
        pltpu.SemaphoreType.DMA,
    ]

    @functools.partial(
        pl.kernel, out_type=out_types, mesh=mesh, scratch_types=scratch,
        compiler_params=pltpu.CompilerParams(needs_layout_passes=False))
    def conv2(h_hbm, srcl_hbm, lidxl_hbm, cnts_hbm, agg_hbm,
              ssrc, slidx, accA, rows0, rows1, cntv,
              g_sem0, g_sem1):
        tid = lax.axis_index("s") * 2 + lax.axis_index("c")
        iota = lax.iota(jnp.int32, 16)
        pltpu.sync_copy(cnts_hbm.at[tid], cntv)
        cv = cntv[pl.ds(0, 16)]
        rows = (rows0, rows1)
        gsems = (g_sem0, g_sem1)

        def do_block(blk, _):
            gb = tid * 4 + blk
            pltpu.sync_copy(srcl_hbm.at[gb], ssrc.at[pl.ds(0, SUBCAP)])
            pltpu.sync_copy(lidxl_hbm.at[gb], slidx.at[pl.ds(0, SUBCAP)])
            cs = jnp.sum(jnp.where(iota == blk, cv, 0))
            _process_block(h_hbm, agg_hbm, ssrc, slidx, (accA,),
                           rows, gsems, cs, gb, packed=True, gbatch=GB,
                           chunks=HD // 16)
            return 0
        lax.fori_loop(0, 4, do_block, 0)

    return conv2


def _dense1(x, agg, wt, cwt, b2d):
    R = 512

    def body(x_ref, agg_ref, w_ref, cw_ref, b_ref, o_ref, op_ref):
        a = jnp.dot(x_ref[...], w_ref[...],
                    preferred_element_type=jnp.float32)
        for c in range(C):
            w = agg_ref[c][:, :HD1]
            low = lax.bitcast_convert_type(w << 16, jnp.float32)
            high = lax.bitcast_convert_type(w & ~0xFFFF, jnp.float32)
            a = a + jnp.dot(low, cw_ref[c][:HD1],
                            preferred_element_type=jnp.float32)
            a = a + jnp.dot(high, cw_ref[c][HD1:],
                            preferred_element_type=jnp.float32)
        h = jnp.maximum(a + b_ref[0:1, :], 0.0)
        o_ref[...] = h
        u0 = lax.bitcast_convert_type(h[:, :HD], jnp.int32)
        u1 = lax.bitcast_convert_type(h[:, HD:], jnp.int32)
        r0 = (u0 + 0x7FFF + ((u0 >> 16) & 1)) >> 16
        r1 = (u1 + 0x7FFF + ((u1 >> 16) & 1)) >> 16
        op_ref[...] = (r0 & 0xFFFF) | (r1 << 16)

    return pl.pallas_call(
        body,
        grid=(NPAD // R,),
        in_specs=[
            pl.BlockSpec((R, 128), lambda i: (i, 0)),
            pl.BlockSpec((C, R, 128), lambda i: (0, i, 0)),
            pl.BlockSpec((128, 256), lambda i: (0, 0)),
            pl.BlockSpec((C, 128, 256), lambda i: (0, 0, 0)),
            pl.BlockSpec((8, 256), lambda i: (0, 0)),
        ],
        out_specs=[pl.BlockSpec((R, 256), lambda i: (i, 0)),
                   pl.BlockSpec((R, HD), lambda i: (i, 0))],
        out_shape=[jax.ShapeDtypeStruct((NPAD, 256), jnp.float32),
                   jax.ShapeDtypeStruct((NPAD, HD), jnp.int32)],
    )(x, agg, wt, cwt, b2d)


def _dense2(x, agg, wt, cwt, b2d):
    R = 512

    def body(x_ref, agg_ref, w_ref, cw_ref, b_ref, o_ref):
        a = jnp.dot(x_ref[...], w_ref[...],
                    preferred_element_type=jnp.float32)
        for c in range(C):
            w = agg_ref[c]
            low = lax.bitcast_convert_type(w << 16, jnp.float32)
            high = lax.bitcast_convert_type(w & ~0xFFFF, jnp.float32)
            a = a + jnp.dot(low, cw_ref[c][:HD],
                            preferred_element_type=jnp.float32)
            a = a + jnp.dot(high, cw_ref[c][HD:],
                            preferred_element_type=jnp.float32)
        o_ref[...] = jax.nn.sigmoid(a + b_ref[0:1, :] - 10.0)

    return pl.pallas_call(
        body,
        grid=(NPAD // R,),
        in_specs=[
            pl.BlockSpec((R, 256), lambda i: (i, 0)),
            pl.BlockSpec((C, R, HD), lambda i: (0, i, 0)),
            pl.BlockSpec((256, 128), lambda i: (0, 0)),
            pl.BlockSpec((C, 256, 128), lambda i: (0, 0, 0)),
            pl.BlockSpec((8, 128), lambda i: (0, 0)),
        ],
        out_specs=pl.BlockSpec((R, 128), lambda i: (i, 0)),
        out_shape=jax.ShapeDtypeStruct((NPAD, 128), jnp.float32),
    )(x, agg, wt, cwt, b2d)


def kernel(x, edge_index, edge_type, conv1_w, conv2_w,
           lin1_w, lin1_b, lin2_w, lin2_b):
    src = edge_index[0].astype(jnp.int32)
    dst = edge_index[1].astype(jnp.int32)
    typ = edge_type.astype(jnp.int32)
    ew = src | ((dst * 4 + typ) << 14)

    xb = lax.bitcast_convert_type(
        x.astype(jnp.bfloat16), jnp.uint16).astype(jnp.int32)
    x_p = jnp.pad(xb[:, :HD1] | (xb[:, HD1:] << 16), ((0, 0), (0, HD1)))
    agg1f, srcl, lidxl, cnts = _build_conv1()(x_p, ew)
    agg1 = agg1f.reshape(C, NPAD, HD)

    x_pad = jnp.pad(x, ((0, NPAD - N), (0, 0)))
    b1 = jnp.broadcast_to(lin1_b[None, :], (8, 256))
    h1, h1p = _dense1(x_pad, agg1, lin1_w.T,
                      jnp.transpose(conv1_w, (0, 2, 1)), b1)

    agg2f = _build_conv2()(h1p, srcl, lidxl, cnts)
    agg2 = agg2f.reshape(C, NPAD, HD)

    b2 = jnp.broadcast_to(lin2_b[None, :], (8, 128))
    out = _dense2(h1, agg2, lin2_w.T, jnp.transpose(conv2_w, (0, 2, 1)), b2)
    return out[:N]
